# baseline (speedup 1.0000x reference)
import functools
import jax
import jax.numpy as jnp
from jax import lax
from jax.experimental import pallas as pl
from jax.experimental.pallas import tpu as pltpu
from jax.experimental.pallas import tpu_sc as plsc

NC = 2
NS = 16
L = 16


def _k1_body(x_ref, w_ref, a1_ref, a2_ref, keys_ref, s1_ref, s2_ref):
    x = x_ref[...]
    keys = jnp.dot(x, w_ref[...], preferred_element_type=jnp.float32)
    keys_ref[...] = keys
    dn = (((0,), (1,)), ((), ()))
    s1_ref[...] = lax.dot_general(a1_ref[...], keys, dn,
                                  preferred_element_type=jnp.float32)
    s2_ref[...] = lax.dot_general(a2_ref[...], keys, dn,
                                  preferred_element_type=jnp.float32)


def _k1(x, wcat, a1blk, a2blk):
    n, d = x.shape
    do = wcat.shape[1]
    h = a1blk.shape[1]
    return pl.pallas_call(
        _k1_body,
        out_shape=[
            jax.ShapeDtypeStruct((n, do), jnp.float32),
            jax.ShapeDtypeStruct((h, n), jnp.float32),
            jax.ShapeDtypeStruct((h, n), jnp.float32),
        ],
    )(x, wcat, a1blk, a2blk)


def _k3_body(keys_ref, t_ref, s_ref, b_ref, out_ref):
    dn = (((0,), (0,)), ((), ()))
    scale = lax.dot_general(t_ref[...], s_ref[...], dn,
                            preferred_element_type=jnp.float32)
    out_ref[...] = jnp.maximum(keys_ref[...] * scale + b_ref[...], 0.0)


def _k3(keys, t_hn, sel, brow):
    n, do = keys.shape
    return pl.pallas_call(
        _k3_body,
        out_shape=jax.ShapeDtypeStruct((n, do), jnp.float32),
    )(keys, t_hn, sel, brow)


R_HIST = 61440 * NS
ZCH = 3840
CH = 8192
KEYSPACE = 4096 * 4096


def _k2a_kernel(e_total):
    et = e_total // NS
    rows = et // 128
    npass = -(-KEYSPACE // (NC * R_HIST))
    mesh = plsc.VectorSubcoreMesh(core_axis_name="c", subcore_axis_name="s",
                                  num_cores=NC, num_subcores=NS)

    @functools.partial(
        pl.kernel,
        out_type=[jax.ShapeDtypeStruct((e_total,), jnp.float32),
                  jax.ShapeDtypeStruct((e_total,), jnp.float32),
                  jax.ShapeDtypeStruct((e_total,), jnp.int32)],
        mesh=mesh,
        compiler_params=pltpu.CompilerParams(needs_layout_passes=False),
        scratch_types=[
            pltpu.VMEM((et,), jnp.int32),
            pltpu.VMEM((CH,), jnp.int32),
            pltpu.VMEM((CH,), jnp.float32),
            pltpu.VMEM((CH,), jnp.int32),
            pltpu.VMEM((CH,), jnp.float32),
            pltpu.VMEM((et,), jnp.float32),
            pltpu.VMEM((ZCH,), jnp.float32),
            pltpu.VMEM_SHARED((R_HIST,), jnp.float32),
        ],
    )
    def k2a(src_hbm, dst_hbm, m0_hbm, m1_hbm, kp_hbm,
            kv, idx2, val2, qidx2, gath2, macc, zv, hist):
        c = lax.axis_index("c")
        s = lax.axis_index("s")
        base = s * et

        pltpu.sync_copy(src_hbm.at[pl.ds(base, et)], kv)
        pltpu.sync_copy(dst_hbm.at[pl.ds(base, CH)], idx2)
        pltpu.sync_copy(dst_hbm.at[pl.ds(base + CH, et - CH)],
                        qidx2.at[pl.ds(0, et - CH)])

        def _init(i, _):
            sl = pl.ds(i * L, L)
            sv = kv[sl]
            dv = jnp.where(i * L < CH, 0, 0)
            del dv
            return 0
        del _init
        def _init1(i, _):
            sl = pl.ds(i * L, L)
            kv[sl] = (kv[sl] << 12) | idx2[sl]
            macc[sl] = jnp.zeros((L,), jnp.float32)
            return 0
        lax.fori_loop(0, CH // L, _init1, 0)

        def _init2(i, _):
            gsl = pl.ds(CH + i * L, L)
            sl = pl.ds(i * L, L)
            kv[gsl] = (kv[gsl] << 12) | qidx2[sl]
            macc[gsl] = jnp.zeros((L,), jnp.float32)
            return 0
        lax.fori_loop(0, (et - CH) // L, _init2, 0)

        def _zzero(i, _):
            zv[pl.ds(i * L, L)] = jnp.zeros((L,), jnp.float32)
            return 0
        lax.fori_loop(0, ZCH // L, _zzero, 0)

        def _pass(p, _):
            lo = (NC * p + c) * R_HIST
            needed = jnp.clip(KEYSPACE - lo, 0, R_HIST)
            for j in range(R_HIST // NS // ZCH):
                off = s * (R_HIST // NS) + j * ZCH

                @pl.when(off < needed)
                def _():
                    pltpu.sync_copy(zv, hist.at[pl.ds(off, ZCH)])
            plsc.subcore_barrier()

            def _sc_chunk(q, _):
                def _build(i, _):
                    gsl = pl.ds(q * CH + i * L, L)
                    csl = pl.ds(i * L, L)
                    k16 = kv[gsl]
                    inr = (k16 >= lo) & (k16 < lo + R_HIST)
                    idx2[csl] = jnp.where(inr, k16 - lo, 0)
                    val2[csl] = jnp.where(inr, 1.0, 0.0)
                    return 0
                lax.fori_loop(0, CH // L, _build, 0)
                pltpu.sync_copy(val2, hist.at[idx2], add=True)
                return 0
            lax.fori_loop(0, et // CH, _sc_chunk, 0)
            plsc.subcore_barrier()

            def _g_chunk(q, _):
                def _buildq(i, _):
                    gsl = pl.ds(q * CH + i * L, L)
                    csl = pl.ds(i * L, L)
                    kk = kv[gsl]
                    rk16 = ((kk & 4095) << 12) | (kk >> 12)
                    inq = (rk16 >= lo) & (rk16 < lo + R_HIST)
                    qidx2[csl] = jnp.where(inq, rk16 - lo, 0)
                    return 0
                lax.fori_loop(0, CH // L, _buildq, 0)
                pltpu.sync_copy(hist.at[qidx2], gath2)

                def _accum(i, _):
                    gsl = pl.ds(q * CH + i * L, L)
                    csl = pl.ds(i * L, L)
                    kk = kv[gsl]
                    rk16 = ((kk & 4095) << 12) | (kk >> 12)
                    inq = (rk16 >= lo) & (rk16 < lo + R_HIST)
                    g = gath2[csl]
                    macc[gsl] = macc[gsl] + jnp.where(inq, g, 0.0)
                    return 0
                lax.fori_loop(0, CH // L, _accum, 0)
                return 0
            lax.fori_loop(0, et // CH, _g_chunk, 0)
            plsc.subcore_barrier()
            return 0
        lax.fori_loop(0, npass, _pass, 0)

        @pl.when(c == 0)
        def _():
            pltpu.sync_copy(macc, m0_hbm.at[pl.ds(base, et)])
            pltpu.sync_copy(kv, kp_hbm.at[pl.ds(base, et)])

        @pl.when(c == 1)
        def _():
            pltpu.sync_copy(macc, m1_hbm.at[pl.ds(base, et)])

    return k2a


STRIDE = 4097


def _k2b_kernel(e_total, n_nodes, n_heads):
    et = e_total // NS
    ns_slice = n_nodes // NS
    acc_words = -(-((L - 1) * STRIDE + n_nodes) // L) * L
    hpc = n_heads // NC
    mesh = plsc.VectorSubcoreMesh(core_axis_name="c", subcore_axis_name="s",
                                  num_cores=NC, num_subcores=NS)

    @functools.partial(
        pl.kernel,
        out_type=jax.ShapeDtypeStruct((n_heads, n_nodes), jnp.float32),
        mesh=mesh,
        compiler_params=pltpu.CompilerParams(needs_layout_passes=False),
        scratch_types=[
            pltpu.VMEM((et,), jnp.int32),
            pltpu.VMEM((et,), jnp.float32),
            pltpu.VMEM((n_nodes,), jnp.float32),
            pltpu.VMEM((n_nodes,), jnp.float32),
            pltpu.VMEM((n_nodes,), jnp.float32),
            pltpu.VMEM((n_nodes,), jnp.float32),
            pltpu.VMEM((L, ns_slice), jnp.float32),
            pltpu.VMEM((acc_words,), jnp.float32),
            pltpu.VMEM_SHARED((NS, n_nodes), jnp.float32),
            pltpu.VMEM_SHARED((n_nodes,), jnp.float32),
        ],
    )
    def k2b(kp_hbm, m0_hbm, m1_hbm, s1_hbm, s2_hbm, t_hbm,
            kv, mv, s1v, s2v, dnv, prt, rst, acc, shp, shr):
        c = lax.axis_index("c")
        s = lax.axis_index("s")
        base = s * et
        lane = lax.iota(jnp.int32, L) * STRIDE

        pltpu.sync_copy(kp_hbm.at[pl.ds(base, et)], kv)

        def _zacc(i, _):
            acc[pl.ds(i * L, L)] = jnp.zeros((L,), jnp.float32)
            return 0

        def _zero_acc():
            lax.fori_loop(0, acc_words // L, _zacc, 0)

        pltpu.sync_copy(m0_hbm.at[pl.ds(base, et)], mv)
        pltpu.sync_copy(m1_hbm.at[pl.ds(base, et)], acc.at[pl.ds(0, et)])

        def _madd(i, _):
            sl = pl.ds(i * L, L)
            mv[sl] = mv[sl] + acc[sl]
            return 0
        lax.fori_loop(0, et // L, _madd, 0)

        for hh in range(hpc):
            h = c * hpc + hh
            pltpu.sync_copy(s1_hbm.at[h], s1v)
            pltpu.sync_copy(s2_hbm.at[h], s2v)
            _zero_acc()

            def _p1(i, _):
                sl = pl.ds(i * L, L)
                kk = kv[sl]
                s16 = kk >> 12
                d16 = kk & 4095
                lg = plsc.load_gather(s1v, [s16]) + plsc.load_gather(s2v, [d16])
                msk = jnp.exp(lg)
                plsc.addupdate_scatter(acc, [lane + s16], msk)
                return 0
            lax.fori_loop(0, et // L, _p1, 0)

            def _red(i, _):
                b2 = i * L
                v = acc[pl.ds(b2, L)]
                for l in range(1, L):
                    v = v + acc[pl.ds(l * STRIDE + b2, L)]
                prt[pl.ds(b2, L)] = v
                return 0
            lax.fori_loop(0, n_nodes // L, _red, 0)

            pltpu.sync_copy(prt, shp.at[s])
            plsc.subcore_barrier()
            pltpu.sync_copy(shp.at[:, pl.ds(s * ns_slice, ns_slice)], rst)

            def _red2(i, _):
                b2 = i * L
                v = rst[0, pl.ds(b2, L)]
                for l in range(1, L):
                    v = v + rst[l, pl.ds(b2, L)]
                prt[pl.ds(b2, L)] = v
                return 0
            lax.fori_loop(0, ns_slice // L, _red2, 0)
            pltpu.sync_copy(prt.at[pl.ds(0, ns_slice)],
                            shr.at[pl.ds(s * ns_slice, ns_slice)])
            plsc.subcore_barrier()
            pltpu.sync_copy(shr, dnv)

            _zero_acc()

            def _p2(i, _):
                sl = pl.ds(i * L, L)
                kk = kv[sl]
                s16 = kk >> 12
                d16 = kk & 4095
                lg = plsc.load_gather(s1v, [s16]) + plsc.load_gather(s2v, [d16])
                msk = jnp.exp(lg)
                dg = plsc.load_gather(dnv, [s16])
                cf = jnp.where(dg == 0.0, 0.0,
                               msk / jnp.where(dg == 0.0, 1.0, dg))
                mm = mv[sl]
                w = jnp.where(mm == 0.0, 0.0, cf * mm)
                plsc.addupdate_scatter(acc, [lane + s16], w)
                plsc.addupdate_scatter(acc, [lane + d16], w)
                return 0
            lax.fori_loop(0, et // L, _p2, 0)

            lax.fori_loop(0, n_nodes // L, _red, 0)
            pltpu.sync_copy(prt, shp.at[s])
            plsc.subcore_barrier()
            pltpu.sync_copy(shp.at[:, pl.ds(s * ns_slice, ns_slice)], rst)
            lax.fori_loop(0, ns_slice // L, _red2, 0)
            pltpu.sync_copy(prt.at[pl.ds(0, ns_slice)],
                            t_hbm.at[h, pl.ds(s * ns_slice, ns_slice)])
            plsc.subcore_barrier()

    return k2b


@jax.jit
def kernel(node_features, edge_list, kernels, att_kernels, biases):
    n, d = node_features.shape
    h, _, hd = kernels.shape
    e = edge_list.shape[0]
    do = h * hd

    src = edge_list[:, 0]
    dst = edge_list[:, 1]
    wcat = jnp.transpose(kernels, (1, 0, 2)).reshape(d, do)
    a1 = att_kernels[:, :hd, 0]
    a2 = att_kernels[:, hd:, 0]
    eye = jnp.eye(h, dtype=jnp.float32)
    a1blk = (eye[:, None, :] * a1[:, :, None]).reshape(do, h)
    a2blk = (eye[:, None, :] * a2[:, :, None]).reshape(do, h)
    sel = jnp.repeat(eye, hd, axis=1)
    brow = biases.reshape(1, do)

    keys, s1, s2 = _k1(node_features, wcat, a1blk, a2blk)
    m0, m1, kp = _k2a_kernel(e)(src, dst)
    t_hn = _k2b_kernel(e, n, h)(kp, m0, m1, s1, s2)
    return _k3(keys, t_hn, sel, brow)


# speedup vs baseline: 5.4507x; 5.4507x over previous
import functools
import jax
import jax.numpy as jnp
from jax import lax
from jax.experimental import pallas as pl
from jax.experimental.pallas import tpu as pltpu
from jax.experimental.pallas import tpu_sc as plsc

NC = 2
NS = 16
L = 16


def _k1_body(x_ref, w_ref, a1_ref, a2_ref, keys_ref, s1_ref, s2_ref):
    x = x_ref[...]
    keys = jnp.dot(x, w_ref[...], preferred_element_type=jnp.float32)
    keys_ref[...] = keys
    dn = (((0,), (1,)), ((), ()))
    s1_ref[...] = lax.dot_general(a1_ref[...], keys, dn,
                                  preferred_element_type=jnp.float32)
    s2_ref[...] = lax.dot_general(a2_ref[...], keys, dn,
                                  preferred_element_type=jnp.float32)


def _k1(x, wcat, a1blk, a2blk):
    n, d = x.shape
    do = wcat.shape[1]
    h = a1blk.shape[1]
    return pl.pallas_call(
        _k1_body,
        out_shape=[
            jax.ShapeDtypeStruct((n, do), jnp.float32),
            jax.ShapeDtypeStruct((h, n), jnp.float32),
            jax.ShapeDtypeStruct((h, n), jnp.float32),
        ],
    )(x, wcat, a1blk, a2blk)


def _k3_body(keys_ref, t_ref, s_ref, b_ref, out_ref):
    dn = (((0,), (0,)), ((), ()))
    scale = lax.dot_general(t_ref[...], s_ref[...], dn,
                            preferred_element_type=jnp.float32)
    out_ref[...] = jnp.maximum(keys_ref[...] * scale + b_ref[...], 0.0)


def _k3(keys, t_hn, sel, brow):
    n, do = keys.shape
    return pl.pallas_call(
        _k3_body,
        out_shape=jax.ShapeDtypeStruct((n, do), jnp.float32),
    )(keys, t_hn, sel, brow)


R_HIST = 63360 * NS
R_COV = R_HIST - 16
ZCH = 1920
SCH = 2048
KEYSPACE = 4096 * 4096


def _k2a_kernel(e_total):
    et = e_total // NS
    npass = -(-KEYSPACE // (NC * R_COV))
    nz = R_HIST // NS // ZCH
    mesh = plsc.VectorSubcoreMesh(core_axis_name="c", subcore_axis_name="s",
                                  num_cores=NC, num_subcores=NS)

    @functools.partial(
        pl.kernel,
        out_type=[jax.ShapeDtypeStruct((e_total,), jnp.float32),
                  jax.ShapeDtypeStruct((e_total,), jnp.float32),
                  jax.ShapeDtypeStruct((e_total,), jnp.int32)],
        mesh=mesh,
        compiler_params=pltpu.CompilerParams(needs_layout_passes=False),
        scratch_types=[
            pltpu.VMEM((et,), jnp.int32),
            pltpu.VMEM((et,), jnp.float32),
            pltpu.VMEM((et + L,), jnp.int32),
            pltpu.VMEM((et + L,), jnp.float32),
            pltpu.VMEM((ZCH,), jnp.float32),
            pltpu.VMEM_SHARED((R_HIST,), jnp.float32),
            pltpu.SemaphoreType.DMA,
        ],
    )
    def k2a(src_hbm, dst_hbm, m0_hbm, m1_hbm, kp_hbm,
            kv, macc, clist, gbuf, zv, hist, sem):
        c = lax.axis_index("c")
        s = lax.axis_index("s")
        base = s * et

        pltpu.sync_copy(src_hbm.at[pl.ds(base, et)], kv)
        pltpu.sync_copy(dst_hbm.at[pl.ds(base, et)], clist.at[pl.ds(0, et)])

        def _initk(i, _):
            sl = pl.ds(i * L, L)
            kv[sl] = (kv[sl] << 12) | clist[sl]
            return 0
        lax.fori_loop(0, et // L, _initk, 0)

        def _initz(i, _):
            zv[pl.ds(i * L, L)] = jnp.zeros((L,), jnp.float32)
            return 0
        lax.fori_loop(0, ZCH // L, _initz, 0)

        dummy = jnp.full((L,), R_COV, jnp.int32)

        def _pass(p, _):
            lo = (NC * p + c) * R_COV
            descs = [
                pltpu.async_copy(
                    zv, hist_slice, sem)
                for hist_slice in [
                    hist.at[pl.ds(s * (R_HIST // NS) + j * ZCH, ZCH)]
                    for j in range(nz)]
            ]

            def _ones(i, _):
                gbuf[pl.ds(i * L, L)] = jnp.full((L,), 1.0, jnp.float32)
                return 0
            lax.fori_loop(0, SCH // L, _ones, 0)

            def _bk(i, cnt):
                sl = pl.ds(i * L, L)
                k16 = kv[sl]
                inr = (k16 >= lo) & (k16 < lo + R_COV)
                plsc.store_compressed(clist.at[pl.ds(cnt, L)], k16 - lo, mask=inr)
                return cnt + jnp.max(plsc.all_reduce_population_count(inr))
            cnt = lax.fori_loop(0, et // L, _bk, jnp.int32(0))
            cnt2 = (cnt + SCH - 1) & ~jnp.int32(SCH - 1)

            def _padk(j, _):
                clist[pl.ds(cnt + j * L, L)] = dummy
                return 0
            lax.fori_loop(0, (cnt2 - cnt + L - 1) // L, _padk, 0)

            for d in descs:
                d.wait()
            plsc.subcore_barrier()

            def _scat(j, _):
                pltpu.sync_copy(gbuf.at[pl.ds(0, SCH)],
                                hist.at[clist.at[pl.ds(j * SCH, SCH)]],
                                add=True)
                return 0
            lax.fori_loop(0, cnt2 // SCH, _scat, 0)

            def _bq(i, qcnt):
                sl = pl.ds(i * L, L)
                kk = kv[sl]
                rk16 = ((kk & 4095) << 12) | (kk >> 12)
                inq = (rk16 >= lo) & (rk16 < lo + R_COV)
                plsc.store_compressed(clist.at[pl.ds(qcnt, L)], rk16 - lo, mask=inq)
                return qcnt + jnp.max(plsc.all_reduce_population_count(inq))
            qcnt = lax.fori_loop(0, et // L, _bq, jnp.int32(0))
            qcnt2 = (qcnt + SCH - 1) & ~jnp.int32(SCH - 1)

            def _padq(j, _):
                clist[pl.ds(qcnt + j * L, L)] = dummy
                return 0
            lax.fori_loop(0, (qcnt2 - qcnt + L - 1) // L, _padq, 0)
            plsc.subcore_barrier()

            def _gat(j, _):
                pltpu.sync_copy(hist.at[clist.at[pl.ds(j * SCH, SCH)]],
                                gbuf.at[pl.ds(j * SCH, SCH)])
                return 0
            lax.fori_loop(0, qcnt2 // SCH, _gat, 0)

            def _cons(i, qq):
                sl = pl.ds(i * L, L)
                kk = kv[sl]
                rk16 = ((kk & 4095) << 12) | (kk >> 12)
                inq = (rk16 >= lo) & (rk16 < lo + R_COV)
                g = plsc.load_expanded(gbuf.at[pl.ds(qq, L)], mask=inq)
                macc[sl] = jnp.where(inq, g, macc[sl])
                return qq + jnp.max(plsc.all_reduce_population_count(inq))
            lax.fori_loop(0, et // L, _cons, jnp.int32(0))
            plsc.subcore_barrier()
            return 0
        lax.fori_loop(0, npass, _pass, 0)

        @pl.when(c == 0)
        def _():
            pltpu.sync_copy(macc, m0_hbm.at[pl.ds(base, et)])
            pltpu.sync_copy(kv, kp_hbm.at[pl.ds(base, et)])

        @pl.when(c == 1)
        def _():
            pltpu.sync_copy(macc, m1_hbm.at[pl.ds(base, et)])

    return k2a


STRIDE = 4097


def _k2b_kernel(e_total, n_nodes, n_heads):
    et = e_total // NS
    ns_slice = n_nodes // NS
    acc_words = -(-((L - 1) * STRIDE + n_nodes) // L) * L
    hpc = n_heads // NC
    mesh = plsc.VectorSubcoreMesh(core_axis_name="c", subcore_axis_name="s",
                                  num_cores=NC, num_subcores=NS)

    @functools.partial(
        pl.kernel,
        out_type=jax.ShapeDtypeStruct((n_heads, n_nodes), jnp.float32),
        mesh=mesh,
        compiler_params=pltpu.CompilerParams(needs_layout_passes=False),
        scratch_types=[
            pltpu.VMEM((et,), jnp.int32),
            pltpu.VMEM((et,), jnp.float32),
            pltpu.VMEM((n_nodes,), jnp.float32),
            pltpu.VMEM((n_nodes,), jnp.float32),
            pltpu.VMEM((n_nodes,), jnp.float32),
            pltpu.VMEM((n_nodes,), jnp.float32),
            pltpu.VMEM((L, ns_slice), jnp.float32),
            pltpu.VMEM((acc_words,), jnp.float32),
            pltpu.VMEM_SHARED((NS, n_nodes), jnp.float32),
            pltpu.VMEM_SHARED((n_nodes,), jnp.float32),
        ],
    )
    def k2b(kp_hbm, m0_hbm, m1_hbm, s1_hbm, s2_hbm, t_hbm,
            kv, mv, s1v, s2v, dnv, prt, rst, acc, shp, shr):
        c = lax.axis_index("c")
        s = lax.axis_index("s")
        base = s * et
        lane = lax.iota(jnp.int32, L) * STRIDE

        pltpu.sync_copy(kp_hbm.at[pl.ds(base, et)], kv)

        def _zacc(i, _):
            acc[pl.ds(i * L, L)] = jnp.zeros((L,), jnp.float32)
            return 0

        def _zero_acc():
            lax.fori_loop(0, acc_words // L, _zacc, 0)

        pltpu.sync_copy(m0_hbm.at[pl.ds(base, et)], mv)
        pltpu.sync_copy(m1_hbm.at[pl.ds(base, et)], acc.at[pl.ds(0, et)])

        def _madd(i, _):
            sl = pl.ds(i * L, L)
            mv[sl] = mv[sl] + acc[sl]
            return 0
        lax.fori_loop(0, et // L, _madd, 0)

        for hh in range(hpc):
            h = c * hpc + hh
            pltpu.sync_copy(s1_hbm.at[h], s1v)
            pltpu.sync_copy(s2_hbm.at[h], s2v)
            _zero_acc()

            def _p1(i, _):
                sl = pl.ds(i * L, L)
                kk = kv[sl]
                s16 = kk >> 12
                d16 = kk & 4095
                lg = plsc.load_gather(s1v, [s16]) + plsc.load_gather(s2v, [d16])
                msk = jnp.exp(lg)
                plsc.addupdate_scatter(acc, [lane + s16], msk)
                return 0
            lax.fori_loop(0, et // L, _p1, 0)

            def _red(i, _):
                b2 = i * L
                v = acc[pl.ds(b2, L)]
                for l in range(1, L):
                    v = v + acc[pl.ds(l * STRIDE + b2, L)]
                prt[pl.ds(b2, L)] = v
                return 0
            lax.fori_loop(0, n_nodes // L, _red, 0)

            pltpu.sync_copy(prt, shp.at[s])
            plsc.subcore_barrier()
            pltpu.sync_copy(shp.at[:, pl.ds(s * ns_slice, ns_slice)], rst)

            def _red2(i, _):
                b2 = i * L
                v = rst[0, pl.ds(b2, L)]
                for l in range(1, L):
                    v = v + rst[l, pl.ds(b2, L)]
                prt[pl.ds(b2, L)] = v
                return 0
            lax.fori_loop(0, ns_slice // L, _red2, 0)
            pltpu.sync_copy(prt.at[pl.ds(0, ns_slice)],
                            shr.at[pl.ds(s * ns_slice, ns_slice)])
            plsc.subcore_barrier()
            pltpu.sync_copy(shr, dnv)

            _zero_acc()

            def _p2(i, _):
                sl = pl.ds(i * L, L)
                kk = kv[sl]
                s16 = kk >> 12
                d16 = kk & 4095
                lg = plsc.load_gather(s1v, [s16]) + plsc.load_gather(s2v, [d16])
                msk = jnp.exp(lg)
                dg = plsc.load_gather(dnv, [s16])
                cf = jnp.where(dg == 0.0, 0.0,
                               msk / jnp.where(dg == 0.0, 1.0, dg))
                mm = mv[sl]
                w = jnp.where(mm == 0.0, 0.0, cf * mm)
                plsc.addupdate_scatter(acc, [lane + s16], w)
                plsc.addupdate_scatter(acc, [lane + d16], w)
                return 0
            lax.fori_loop(0, et // L, _p2, 0)

            lax.fori_loop(0, n_nodes // L, _red, 0)
            pltpu.sync_copy(prt, shp.at[s])
            plsc.subcore_barrier()
            pltpu.sync_copy(shp.at[:, pl.ds(s * ns_slice, ns_slice)], rst)
            lax.fori_loop(0, ns_slice // L, _red2, 0)
            pltpu.sync_copy(prt.at[pl.ds(0, ns_slice)],
                            t_hbm.at[h, pl.ds(s * ns_slice, ns_slice)])
            plsc.subcore_barrier()

    return k2b


@jax.jit
def kernel(node_features, edge_list, kernels, att_kernels, biases):
    n, d = node_features.shape
    h, _, hd = kernels.shape
    e = edge_list.shape[0]
    do = h * hd

    src = edge_list[:, 0]
    dst = edge_list[:, 1]
    wcat = jnp.transpose(kernels, (1, 0, 2)).reshape(d, do)
    a1 = att_kernels[:, :hd, 0]
    a2 = att_kernels[:, hd:, 0]
    eye = jnp.eye(h, dtype=jnp.float32)
    a1blk = (eye[:, None, :] * a1[:, :, None]).reshape(do, h)
    a2blk = (eye[:, None, :] * a2[:, :, None]).reshape(do, h)
    sel = jnp.repeat(eye, hd, axis=1)
    brow = biases.reshape(1, do)

    keys, s1, s2 = _k1(node_features, wcat, a1blk, a2blk)
    m0, m1, kp = _k2a_kernel(e)(src, dst)
    t_hn = _k2b_kernel(e, n, h)(kp, m0, m1, s1, s2)
    return _k3(keys, t_hn, sel, brow)
